# baseline (speedup 1.0000x reference)
import jax
import jax.numpy as jnp
from jax import lax
from jax.experimental import pallas as pl
from jax.experimental.pallas import tpu as pltpu
from jax.experimental.pallas import tpu_sc as plsc

B = 16384
D = 64
NC = 2
NS = 16
NW = NC * NS
BPW = B // NW
L = 16
NG = BPW // L


def _sc_body(u_ids_hbm, i_ids_hbm, ent_map_hbm, user_hbm, item_hbm, ent_hbm,
             out_hbm, uid_v, iid_v, eid_v, u_rows, i_rows, e_rows, out_v,
             sem_u, sem_i, sem_e):
    wid = lax.axis_index("s") * NC + lax.axis_index("c")
    base = wid * BPW

    pltpu.sync_copy(u_ids_hbm.at[pl.ds(base, BPW)], uid_v)
    pltpu.sync_copy(i_ids_hbm.at[pl.ds(base, BPW)], iid_v)
    cp_u = pltpu.async_copy(user_hbm.at[uid_v], u_rows, sem_u)
    cp_i = pltpu.async_copy(item_hbm.at[iid_v], i_rows, sem_i)
    pltpu.async_copy(ent_map_hbm.at[iid_v], eid_v, sem_e).wait()
    cp_e = pltpu.async_copy(ent_hbm.at[eid_v], e_rows, sem_e)
    cp_u.wait()
    cp_i.wait()
    cp_e.wait()

    lane = lax.iota(jnp.int32, L)

    def group_body(g, _):
        b0 = g * L
        acc = jnp.zeros((L,), jnp.float32)
        for k in range(L):
            b = b0 + k
            p = jnp.zeros((L,), jnp.float32)
            for j in range(D // L):
                sl = pl.ds(j * L, L)
                p = p + u_rows[b, sl] * (i_rows[b, sl] + e_rows[b, sl])
            acc = jnp.where(lane == k, jnp.sum(p), acc)
        out_v[pl.ds(b0, L)] = acc
        return 0

    lax.fori_loop(0, NG, group_body, 0)
    pltpu.sync_copy(out_v, out_hbm.at[pl.ds(base, BPW)])


def kernel(u_ids, i_ids, ent_map, user_emb, item_emb, ent_emb):
    mesh = plsc.VectorSubcoreMesh(core_axis_name="c", subcore_axis_name="s")
    f = pl.kernel(
        _sc_body,
        out_type=jax.ShapeDtypeStruct((B,), jnp.float32),
        mesh=mesh,
        compiler_params=pltpu.CompilerParams(
            needs_layout_passes=False, use_tc_tiling_on_sc=False),
        scratch_types=[
            pltpu.VMEM((BPW,), jnp.int32),
            pltpu.VMEM((BPW,), jnp.int32),
            pltpu.VMEM((BPW,), jnp.int32),
            pltpu.VMEM((BPW, D), jnp.float32),
            pltpu.VMEM((BPW, D), jnp.float32),
            pltpu.VMEM((BPW, D), jnp.float32),
            pltpu.VMEM((BPW,), jnp.float32),
            pltpu.SemaphoreType.DMA,
            pltpu.SemaphoreType.DMA,
            pltpu.SemaphoreType.DMA,
        ],
    )
    return f(u_ids.astype(jnp.int32), i_ids.astype(jnp.int32),
             ent_map.astype(jnp.int32), user_emb, item_emb, ent_emb)
